# baseline (speedup 1.0000x reference)
import functools

import jax
import jax.numpy as jnp
from jax.experimental import pallas as pl
from jax.experimental.pallas import tpu as pltpu

HIDDEN = 16
LANES = 128
NUM_CLASSES = 7
OUT_LANES = 7


def _round_up(x, m):
    return ((x + m - 1) // m) * m


def _slab_offsets(f_pad):
    o_w1 = 0
    o_b1 = _round_up(max(f_pad, 8), 8)
    o_w2 = o_b1 + 8
    o_b2 = o_w2 + LANES
    return o_w1, o_b1, o_w2, o_b2


def _xw_kernel(x_ref, slab_ref, xw_ref, *, num_features):
    w1 = slab_ref[0:num_features, :].astype(jnp.bfloat16)
    xb = x_ref[...].astype(jnp.bfloat16)
    xw_ref[...] = jnp.dot(
        xb, w1, preferred_element_type=jnp.float32).astype(jnp.bfloat16)


def _layer1_kernel(a_ref, xw_ref, slab_ref, hw_ref, *, f_pad):
    _, o_b1, o_w2, _ = _slab_offsets(f_pad)
    b1 = slab_ref[o_b1:o_b1 + 1, :]
    w2 = slab_ref[o_w2:o_w2 + LANES, :].astype(jnp.bfloat16)
    pre1 = jnp.dot(a_ref[...], xw_ref[...],
                   preferred_element_type=jnp.float32)
    h1 = jnp.maximum(pre1 + b1, 0.0).astype(jnp.bfloat16)
    hw_ref[...] = jnp.dot(h1, w2,
                          preferred_element_type=jnp.float32).astype(jnp.bfloat16)


def _layer2_kernel(a_ref, hw_ref, slab_ref, out_ref, *, f_pad):
    _, _, _, o_b2 = _slab_offsets(f_pad)
    b2 = slab_ref[o_b2:o_b2 + 1, :]
    logits = jnp.dot(a_ref[...], hw_ref[...],
                     preferred_element_type=jnp.float32) + b2
    m = jnp.max(logits, axis=1, keepdims=True)
    z = logits - m
    lse = jnp.log(jnp.sum(jnp.exp(z), axis=1, keepdims=True))
    out_ref[...] = (z - lse)[:, :OUT_LANES]


@jax.jit
def _forward(x, a_hat_pad, slab):
    num_nodes, num_features = x.shape
    n_pad = a_hat_pad.shape[0]
    f_pad = _round_up(max(num_features, 1), LANES)

    vmem_limit = 64 * 1024 * 1024 * 3 // 4
    cparams = pltpu.CompilerParams(
        dimension_semantics=("parallel",),
        vmem_limit_bytes=vmem_limit,
    )
    slab_spec = pl.BlockSpec(slab.shape, lambda i: (0, 0))

    tmx = 2048
    xw = pl.pallas_call(
        functools.partial(_xw_kernel, num_features=num_features),
        grid=(n_pad // tmx,),
        in_specs=[
            pl.BlockSpec((tmx, num_features), lambda i: (i, 0)),
            slab_spec,
        ],
        out_specs=pl.BlockSpec((tmx, LANES), lambda i: (i, 0)),
        out_shape=jax.ShapeDtypeStruct((n_pad, LANES), jnp.bfloat16),
        compiler_params=cparams,
    )(x, slab)

    tm = 2048
    grid = (n_pad // tm,)
    a_spec = pl.BlockSpec((tm, n_pad), lambda i: (i, 0))
    skinny_spec = pl.BlockSpec((n_pad, LANES), lambda i: (0, 0))
    hw_out_spec = pl.BlockSpec((tm, LANES), lambda i: (i, 0))

    hw = pl.pallas_call(
        functools.partial(_layer1_kernel, f_pad=f_pad),
        grid=grid,
        in_specs=[a_spec, skinny_spec, slab_spec],
        out_specs=hw_out_spec,
        out_shape=jax.ShapeDtypeStruct((n_pad, LANES), jnp.bfloat16),
        compiler_params=cparams,
    )(a_hat_pad, xw, slab)

    out = pl.pallas_call(
        functools.partial(_layer2_kernel, f_pad=f_pad),
        grid=grid,
        in_specs=[a_spec, skinny_spec, slab_spec],
        out_specs=pl.BlockSpec((tm, OUT_LANES), lambda i: (i, 0)),
        out_shape=jax.ShapeDtypeStruct((n_pad, OUT_LANES), jnp.float32),
        compiler_params=cparams,
    )(a_hat_pad, hw, slab)

    return out


def kernel(x, a_hat_pad, slab):
    return _forward(x, a_hat_pad, slab)


# speedup vs baseline: 1.0394x; 1.0394x over previous
import functools

import jax
import jax.numpy as jnp
from jax.experimental import pallas as pl
from jax.experimental.pallas import tpu as pltpu

HIDDEN = 16
LANES = 128
NUM_CLASSES = 7
OUT_LANES = 7


def _round_up(x, m):
    return ((x + m - 1) // m) * m


def _slab_offsets(f_pad):
    o_w1 = 0
    o_b1 = _round_up(max(f_pad, 8), 8)
    o_w2 = o_b1 + 8
    o_b2 = o_w2 + LANES
    return o_w1, o_b1, o_w2, o_b2


def _xw_kernel(x_ref, slab_ref, xw_ref, *, num_features):
    w1 = slab_ref[0:num_features, :].astype(jnp.bfloat16)
    xb = x_ref[...].astype(jnp.bfloat16)
    xw_ref[...] = jnp.dot(
        xb, w1, preferred_element_type=jnp.float32).astype(jnp.bfloat16)


def _layer1_kernel(a_ref, xw_ref, slab_ref, hw_ref, *, f_pad):
    _, o_b1, o_w2, _ = _slab_offsets(f_pad)
    b1 = slab_ref[o_b1:o_b1 + 1, :]
    w2 = slab_ref[o_w2:o_w2 + LANES, :].astype(jnp.bfloat16)
    pre1 = jnp.dot(a_ref[...], xw_ref[...],
                   preferred_element_type=jnp.float32)
    h1 = jnp.maximum(pre1 + b1, 0.0).astype(jnp.bfloat16)
    hw_ref[...] = jnp.dot(h1, w2,
                          preferred_element_type=jnp.float32).astype(jnp.bfloat16)


def _layer2_kernel(a_ref, hw_ref, slab_ref, out_ref, *, f_pad):
    _, _, _, o_b2 = _slab_offsets(f_pad)
    b2 = slab_ref[o_b2:o_b2 + 1, :]
    logits = jnp.dot(a_ref[...], hw_ref[...],
                     preferred_element_type=jnp.float32) + b2
    m = jnp.max(logits, axis=1, keepdims=True)
    z = logits - m
    lse = jnp.log(jnp.sum(jnp.exp(z), axis=1, keepdims=True))
    out_ref[...] = (z - lse)[:, :OUT_LANES]


@jax.jit
def _forward(x, a_hat_pad, slab):
    num_nodes, num_features = x.shape
    n_pad = a_hat_pad.shape[0]
    f_pad = _round_up(max(num_features, 1), LANES)

    vmem_limit = 64 * 1024 * 1024 * 3 // 4
    cparams = pltpu.CompilerParams(
        dimension_semantics=("parallel",),
        vmem_limit_bytes=vmem_limit,
    )
    slab_spec = pl.BlockSpec(slab.shape, lambda i: (0, 0))

    tmx = 1024
    xw = pl.pallas_call(
        functools.partial(_xw_kernel, num_features=num_features),
        grid=(n_pad // tmx,),
        in_specs=[
            pl.BlockSpec((tmx, num_features), lambda i: (i, 0)),
            slab_spec,
        ],
        out_specs=pl.BlockSpec((tmx, LANES), lambda i: (i, 0)),
        out_shape=jax.ShapeDtypeStruct((n_pad, LANES), jnp.bfloat16),
        compiler_params=cparams,
    )(x, slab)

    tm = 1024
    grid = (n_pad // tm,)
    a_spec = pl.BlockSpec((tm, n_pad), lambda i: (i, 0))
    skinny_spec = pl.BlockSpec((n_pad, LANES), lambda i: (0, 0))
    hw_out_spec = pl.BlockSpec((tm, LANES), lambda i: (i, 0))

    hw = pl.pallas_call(
        functools.partial(_layer1_kernel, f_pad=f_pad),
        grid=grid,
        in_specs=[a_spec, skinny_spec, slab_spec],
        out_specs=hw_out_spec,
        out_shape=jax.ShapeDtypeStruct((n_pad, LANES), jnp.bfloat16),
        compiler_params=cparams,
    )(a_hat_pad, xw, slab)

    out = pl.pallas_call(
        functools.partial(_layer2_kernel, f_pad=f_pad),
        grid=grid,
        in_specs=[a_spec, skinny_spec, slab_spec],
        out_specs=pl.BlockSpec((tm, OUT_LANES), lambda i: (i, 0)),
        out_shape=jax.ShapeDtypeStruct((n_pad, OUT_LANES), jnp.float32),
        compiler_params=cparams,
    )(a_hat_pad, hw, slab)

    return out


def kernel(x, a_hat_pad, slab):
    return _forward(x, a_hat_pad, slab)


# speedup vs baseline: 1.0424x; 1.0029x over previous
import functools

import jax
import jax.numpy as jnp
from jax.experimental import pallas as pl
from jax.experimental.pallas import tpu as pltpu

HIDDEN = 16
LANES = 128
NUM_CLASSES = 7
OUT_LANES = 7


def _round_up(x, m):
    return ((x + m - 1) // m) * m


def _slab_offsets(f_pad):
    o_w1 = 0
    o_b1 = _round_up(max(f_pad, 8), 8)
    o_w2 = o_b1 + 8
    o_b2 = o_w2 + LANES
    return o_w1, o_b1, o_w2, o_b2


def _xw_kernel(x_ref, slab_ref, xw_ref, *, num_features):
    w1 = slab_ref[0:num_features, :].astype(jnp.bfloat16)
    xb = x_ref[...].astype(jnp.bfloat16)
    xw_ref[...] = jnp.dot(
        xb, w1, preferred_element_type=jnp.float32).astype(jnp.bfloat16)


def _layer1_kernel(a_ref, xw_ref, slab_ref, hw_ref, *, f_pad):
    _, o_b1, o_w2, _ = _slab_offsets(f_pad)
    b1 = slab_ref[o_b1:o_b1 + 1, :]
    w2 = slab_ref[o_w2:o_w2 + LANES, :].astype(jnp.bfloat16)
    pre1 = jnp.dot(a_ref[...], xw_ref[...],
                   preferred_element_type=jnp.float32)
    h1 = jnp.maximum(pre1 + b1, 0.0).astype(jnp.bfloat16)
    hw_ref[...] = jnp.dot(h1, w2,
                          preferred_element_type=jnp.float32).astype(jnp.bfloat16)


def _layer2_kernel(a_ref, hw_ref, slab_ref, out_ref, *, f_pad):
    _, _, _, o_b2 = _slab_offsets(f_pad)
    b2 = slab_ref[o_b2:o_b2 + 1, :]
    logits = jnp.dot(a_ref[...], hw_ref[...],
                     preferred_element_type=jnp.float32) + b2
    m = jnp.max(logits, axis=1, keepdims=True)
    z = logits - m
    lse = jnp.log(jnp.sum(jnp.exp(z), axis=1, keepdims=True))
    out_ref[...] = (z - lse)[:, :OUT_LANES]


@jax.jit
def _forward(x, a_hat_pad, slab):
    num_nodes, num_features = x.shape
    n_pad = a_hat_pad.shape[0]
    f_pad = _round_up(max(num_features, 1), LANES)

    vmem_limit = 64 * 1024 * 1024 * 3 // 4
    cparams = pltpu.CompilerParams(
        dimension_semantics=("parallel",),
        vmem_limit_bytes=vmem_limit,
    )
    slab_spec = pl.BlockSpec(slab.shape, lambda i: (0, 0))
    cparams_seq = pltpu.CompilerParams(
        dimension_semantics=("arbitrary",),
        vmem_limit_bytes=vmem_limit,
    )

    tmx = 1024
    xw = pl.pallas_call(
        functools.partial(_xw_kernel, num_features=num_features),
        grid=(n_pad // tmx,),
        in_specs=[
            pl.BlockSpec((tmx, num_features), lambda i: (i, 0)),
            slab_spec,
        ],
        out_specs=pl.BlockSpec((tmx, LANES), lambda i: (i, 0)),
        out_shape=jax.ShapeDtypeStruct((n_pad, LANES), jnp.bfloat16),
        compiler_params=cparams_seq,
    )(x, slab)

    tm = 1024
    grid = (n_pad // tm,)
    a_spec = pl.BlockSpec((tm, n_pad), lambda i: (i, 0))
    skinny_spec = pl.BlockSpec((n_pad, LANES), lambda i: (0, 0))
    hw_out_spec = pl.BlockSpec((tm, LANES), lambda i: (i, 0))

    hw = pl.pallas_call(
        functools.partial(_layer1_kernel, f_pad=f_pad),
        grid=grid,
        in_specs=[a_spec, skinny_spec, slab_spec],
        out_specs=hw_out_spec,
        out_shape=jax.ShapeDtypeStruct((n_pad, LANES), jnp.bfloat16),
        compiler_params=cparams,
    )(a_hat_pad, xw, slab)

    out = pl.pallas_call(
        functools.partial(_layer2_kernel, f_pad=f_pad),
        grid=grid,
        in_specs=[a_spec, skinny_spec, slab_spec],
        out_specs=pl.BlockSpec((tm, OUT_LANES), lambda i: (i, 0)),
        out_shape=jax.ShapeDtypeStruct((n_pad, OUT_LANES), jnp.float32),
        compiler_params=cparams,
    )(a_hat_pad, hw, slab)

    return out


def kernel(x, a_hat_pad, slab):
    return _forward(x, a_hat_pad, slab)
